# baseline (speedup 1.0000x reference)
import jax
import jax.numpy as jnp
from jax import lax
from jax.experimental import pallas as pl
from jax.experimental.pallas import tpu as pltpu
from jax.experimental.pallas import tpu_sc as plsc

D = 1024
B = 4
T = 8192
NC = 2
NS = 16
NW = NC * NS
TPW = T // NW
P = 8
NCH = TPW // P
NBUF = 3
RPB = B * P


def _body(x_hbm, tok_hbm, pos_hbm, out_hbm, idx_v, tok_buf, pos_buf, *sems):
    sem_g = sems[0:3]
    sem_p = sems[3:6]
    sem_w = sems[6:9]
    wid = lax.axis_index("s") * NC + lax.axis_index("c")
    t0 = wid * TPW
    pltpu.sync_copy(x_hbm.at[wid], idx_v)

    def issue_round(r, p):
        pltpu.async_copy(tok_hbm.at[idx_v.at[r]], tok_buf.at[p], sem_g[p])
        pltpu.async_copy(pos_hbm.at[pl.ds(t0 + r * P, P)], pos_buf.at[p], sem_p[p])

    def wait_round(r, p):
        pltpu.make_async_copy(tok_hbm.at[idx_v.at[r]], tok_buf.at[p], sem_g[p]).wait()
        pltpu.make_async_copy(pos_hbm.at[pl.ds(t0 + r * P, P)], pos_buf.at[p], sem_p[p]).wait()

    def issue_writes(r, p):
        for b in range(B):
            pltpu.async_copy(tok_buf.at[p, pl.ds(b * P, P)],
                             out_hbm.at[pl.ds(b * T + t0 + r * P, P)], sem_w[p])

    def drain_writes(r, p):
        for b in range(B):
            pltpu.make_async_copy(tok_buf.at[p, pl.ds(b * P, P)],
                                  out_hbm.at[pl.ds(b * T + t0 + r * P, P)], sem_w[p]).wait()

    def do_round(r, p, pn):
        @pl.when(r >= 2)
        def _():
            drain_writes(r - 2, pn)

        @pl.when(r + 1 < NCH)
        def _():
            issue_round(r + 1, pn)

        wait_round(r, p)

        @plsc.parallel_loop(0, D // 16, unroll=2)
        def add_i(i):
            sl = pl.ds(i * 16, 16)
            for j in range(P):
                pv = pos_buf[p, j, sl]
                for b in range(B):
                    row = b * P + j
                    tok_buf[p, row, sl] = tok_buf[p, row, sl] + pv
        issue_writes(r, p)

    issue_round(0, 0)

    def outer(o, carry):
        for u in range(NBUF):
            r = NBUF * o + u
            do_round(r, u, (u + 1) % NBUF)
        return carry

    lax.fori_loop(0, (NCH - 2) // NBUF, outer, 0)
    do_round(NCH - 2, (NCH - 2) % NBUF, (NCH - 1) % NBUF)
    do_round(NCH - 1, (NCH - 1) % NBUF, NCH % NBUF)
    for r in range(NCH - 2, NCH):
        drain_writes(r, r % NBUF)


def kernel(x, token_table, pos_table):
    xf = (x.astype(jnp.int32)
          .reshape(B, NW, NCH, P)
          .transpose(1, 2, 0, 3)
          .reshape(NW, NCH, RPB))
    mesh = plsc.VectorSubcoreMesh(core_axis_name="c", subcore_axis_name="s")
    k = pl.kernel(
        _body,
        out_type=jax.ShapeDtypeStruct((B * T, D), jnp.float32),
        mesh=mesh,
        scratch_types=[
            pltpu.VMEM((NCH, RPB), jnp.int32),
            pltpu.VMEM((NBUF, RPB, D), jnp.float32),
            pltpu.VMEM((NBUF, P, D), jnp.float32),
        ] + [pltpu.SemaphoreType.DMA] * 9,
    )
    out = k(xf, token_table, pos_table)
    return out.reshape(B, T, D)


# speedup vs baseline: 1.0402x; 1.0402x over previous
import jax
import jax.numpy as jnp
from jax import lax
from jax.experimental import pallas as pl
from jax.experimental.pallas import tpu as pltpu
from jax.experimental.pallas import tpu_sc as plsc

D = 1024
B = 4
T = 8192
NC = 2
NS = 16
NW = NC * NS
TPW = T // NW
P = 8
NCH = TPW // P
NBUF = 3
RPB = B * P


def _body(x_hbm, tok_hbm, pos_hbm, out_hbm, idx_v, tok_buf, pos_buf, *sems):
    sem_g = sems[0:3]
    sem_p = sems[3:6]
    sem_w = sems[6:9]
    wid = lax.axis_index("s") * NC + lax.axis_index("c")
    t0 = wid * TPW
    pltpu.sync_copy(x_hbm.at[wid], idx_v)

    def issue_round(r, p):
        pltpu.async_copy(tok_hbm.at[idx_v.at[r]], tok_buf.at[p], sem_g[p])
        pltpu.async_copy(pos_hbm.at[pl.ds(t0 + r * P, P)], pos_buf.at[p], sem_p[p])

    def wait_round(r, p):
        pltpu.make_async_copy(tok_hbm.at[idx_v.at[r]], tok_buf.at[p], sem_g[p]).wait()
        pltpu.make_async_copy(pos_hbm.at[pl.ds(t0 + r * P, P)], pos_buf.at[p], sem_p[p]).wait()

    def issue_writes(r, p):
        for b in range(B):
            pltpu.async_copy(tok_buf.at[p, pl.ds(b * P, P)],
                             out_hbm.at[pl.ds(b * T + t0 + r * P, P)], sem_w[p])

    def drain_writes(r, p):
        for b in range(B):
            pltpu.make_async_copy(tok_buf.at[p, pl.ds(b * P, P)],
                                  out_hbm.at[pl.ds(b * T + t0 + r * P, P)], sem_w[p]).wait()

    def do_round(r, p, pn):
        @pl.when(r >= 2)
        def _():
            drain_writes(r - 2, pn)

        @pl.when(r + 1 < NCH)
        def _():
            issue_round(r + 1, pn)

        wait_round(r, p)

        @plsc.parallel_loop(0, D // 16)
        def add_i(i):
            sl = pl.ds(i * 16, 16)
            for j in range(P):
                pv = pos_buf[p, j, sl]
                for b in range(B):
                    row = b * P + j
                    tok_buf[p, row, sl] = tok_buf[p, row, sl] + pv
        issue_writes(r, p)

    issue_round(0, 0)

    def outer(o, carry):
        for u in range(NBUF):
            r = NBUF * o + u
            do_round(r, u, (u + 1) % NBUF)
        return carry

    lax.fori_loop(0, (NCH - 2) // NBUF, outer, 0)
    do_round(NCH - 2, (NCH - 2) % NBUF, (NCH - 1) % NBUF)
    do_round(NCH - 1, (NCH - 1) % NBUF, NCH % NBUF)
    for r in range(NCH - 2, NCH):
        drain_writes(r, r % NBUF)


def kernel(x, token_table, pos_table):
    xf = (x.astype(jnp.int32)
          .reshape(B, NW, NCH, P)
          .transpose(1, 2, 0, 3)
          .reshape(NW, NCH, RPB))
    mesh = plsc.VectorSubcoreMesh(core_axis_name="c", subcore_axis_name="s")
    k = pl.kernel(
        _body,
        out_type=jax.ShapeDtypeStruct((B * T, D), jnp.float32),
        mesh=mesh,
        scratch_types=[
            pltpu.VMEM((NCH, RPB), jnp.int32),
            pltpu.VMEM((NBUF, RPB, D), jnp.float32),
            pltpu.VMEM((NBUF, P, D), jnp.float32),
        ] + [pltpu.SemaphoreType.DMA] * 9,
    )
    out = k(xf, token_table, pos_table)
    return out.reshape(B, T, D)
